# baseline (speedup 1.0000x reference)
import functools

import jax
import jax.numpy as jnp
from jax import lax
from jax.experimental import pallas as pl
from jax.experimental.pallas import tpu as pltpu
from jax.experimental.pallas import tpu_sc as plsc

_MAX_DET = 100
_IOU_THRESH = 0.5
_SCORE_THRESH = 0.05
_L = 16
_CAP = 128
_GATE = 112
_CH = 128
_NEG = -1e9


def _hmax16(w):
    m01 = jnp.maximum(w[0], w[1])
    m23 = jnp.maximum(w[2], w[3])
    m45 = jnp.maximum(w[4], w[5])
    m67 = jnp.maximum(w[6], w[7])
    m89 = jnp.maximum(w[8], w[9])
    mab = jnp.maximum(w[10], w[11])
    mcd = jnp.maximum(w[12], w[13])
    mef = jnp.maximum(w[14], w[15])
    return jnp.maximum(
        jnp.maximum(jnp.maximum(m01, m23), jnp.maximum(m45, m67)),
        jnp.maximum(jnp.maximum(m89, mab), jnp.maximum(mcd, mef)))


def _nms_body(nbuf, nch, s_hbm, o_hbm, x1_hbm, y1_hbm, x2_hbm, y2_hbm,
              out_hbm,
              s_v, o_v, x1_v, y1_v, x2_v, y2_v,
              sx1, sy1, sx2, sy2, sar, ssc,
              fx1, fy1, fx2, fy2, out_v, st):
    @pl.when((lax.axis_index("c") == 0) & (lax.axis_index("s") == 0))
    def _tile0():
        pltpu.sync_copy(s_hbm, s_v)
        pltpu.sync_copy(o_hbm, o_v)
        pltpu.sync_copy(x1_hbm, x1_v)
        pltpu.sync_copy(y1_hbm, y1_v)
        pltpu.sync_copy(x2_hbm, x2_v)
        pltpu.sync_copy(y2_hbm, y2_v)

        lane = lax.iota(jnp.int32, _L)
        lane0 = lane == 0
        zeros = jnp.zeros((_L,), jnp.float32)

        def _sinit(c, _):
            sl = pl.ds(c * _L, _L)
            sx1[sl] = jnp.full((_L,), -4.0, jnp.float32)
            sy1[sl] = jnp.full((_L,), -4.0, jnp.float32)
            sx2[sl] = jnp.full((_L,), -2.0, jnp.float32)
            sy2[sl] = jnp.full((_L,), -2.0, jnp.float32)
            sar[sl] = jnp.full((_L,), 4.0, jnp.float32)
            ssc[sl] = jnp.full((_L,), _NEG, jnp.float32)
            fx1[sl] = zeros
            fy1[sl] = zeros
            fx2[sl] = zeros
            fy2[sl] = zeros
            return 0
        lax.fori_loop(0, _CAP // _L, _sinit, 0)

        st[0] = jnp.int32(0)
        st[1] = jnp.int32(0)
        st[2] = jnp.int32(0)

        def _chunk(b, _):
            @pl.when(st[2] == 0)
            def _run_chunk():
                def _cand(j, c):
                    kept, fc = c
                    i = b * _CH + j
                    sc = s_v[pl.ds(i, _L)][0]
                    idx = o_v[pl.ds(i, _L)][0]
                    bx1 = x1_v[pl.ds(idx, _L)][0]
                    by1 = y1_v[pl.ds(idx, _L)][0]
                    bx2 = x2_v[pl.ds(idx, _L)][0]
                    by2 = y2_v[pl.ds(idx, _L)][0]
                    scv = jnp.full((_L,), sc, jnp.float32)
                    bx1v = jnp.full((_L,), bx1, jnp.float32)
                    by1v = jnp.full((_L,), by1, jnp.float32)
                    bx2v = jnp.full((_L,), bx2, jnp.float32)
                    by2v = jnp.full((_L,), by2, jnp.float32)
                    bav = (bx2v - bx1v) * (by2v - by1v)

                    def _iou_chunk(cc, acc):
                        sl = pl.ds(cc * _L, _L)
                        ltx = jnp.maximum(sx1[sl], bx1v)
                        lty = jnp.maximum(sy1[sl], by1v)
                        rbx = jnp.minimum(sx2[sl], bx2v)
                        rby = jnp.minimum(sy2[sl], by2v)
                        w = jnp.maximum(rbx - ltx, 0.0)
                        h = jnp.maximum(rby - lty, 0.0)
                        inter = w * h
                        iou = inter / (sar[sl] + bav - inter + 1e-9)
                        return jnp.maximum(acc, iou)
                    acc = lax.fori_loop(0, (kept + _L - 1) // _L,
                                        _iou_chunk, zeros)

                    valid = sc > jnp.float32(-1e8)
                    suppressed = _hmax16(acc) > jnp.float32(_IOU_THRESH)
                    keep_it = valid & jnp.logical_not(suppressed) & (kept < _GATE)
                    fill_it = ((jnp.logical_not(valid) | suppressed)
                               & (fc < _MAX_DET))

                    kf = jnp.where(keep_it, jnp.float32(1.0), jnp.float32(0.0))
                    kmask = jnp.where(lane0, jnp.full((_L,), kf, jnp.float32),
                                      zeros) > 0.0
                    for arr, val in ((sx1, bx1v), (sy1, by1v), (sx2, bx2v),
                                     (sy2, by2v), (sar, bav), (ssc, scv)):
                        wnd = arr[pl.ds(kept, _L)]
                        arr[pl.ds(kept, _L)] = jnp.where(kmask, val, wnd)

                    ff = jnp.where(fill_it, jnp.float32(1.0), jnp.float32(0.0))
                    fmask = jnp.where(lane0, jnp.full((_L,), ff, jnp.float32),
                                      zeros) > 0.0
                    for arr, val in ((fx1, bx1v), (fy1, by1v), (fx2, bx2v),
                                     (fy2, by2v)):
                        wnd = arr[pl.ds(fc, _L)]
                        arr[pl.ds(fc, _L)] = jnp.where(fmask, val, wnd)

                    kept2 = kept + jnp.where(keep_it, 1, 0).astype(jnp.int32)
                    fc2 = fc + jnp.where(fill_it, 1, 0).astype(jnp.int32)
                    return (kept2, fc2)

                kept, fc = lax.fori_loop(0, _CH, _cand, (st[0], st[1]))
                st[0] = kept
                st[1] = fc
                lastsc = s_v[pl.ds(b * _CH + _CH - 1, _L)][0]
                done = ((kept >= _MAX_DET)
                        | ((lastsc <= jnp.float32(-1e8)) & (fc >= _MAX_DET)))
                st[2] = jnp.where(done, 1, 0).astype(jnp.int32)
            return 0
        lax.fori_loop(0, nch, _chunk, 0)

        def _bulk(c, _):
            sl = pl.ds(c * _L, _L)
            out_v[0, sl] = sx1[sl]
            out_v[1, sl] = sy1[sl]
            out_v[2, sl] = sx2[sl]
            out_v[3, sl] = sy2[sl]
            out_v[4, sl] = ssc[sl]
            return 0
        lax.fori_loop(0, _CAP // _L, _bulk, 0)

        kstart = jnp.minimum(st[0], jnp.int32(_MAX_DET))

        def _patch(j, _):
            kept = st[0]
            fj = j - kept
            vx1 = fx1[pl.ds(fj, _L)][0]
            vy1 = fy1[pl.ds(fj, _L)][0]
            vx2 = fx2[pl.ds(fj, _L)][0]
            vy2 = fy2[pl.ds(fj, _L)][0]
            for r, val in enumerate((vx1, vy1, vx2, vy2,
                                     jnp.float32(_NEG))):
                wnd = out_v[r, pl.ds(j, _L)]
                out_v[r, pl.ds(j, _L)] = jnp.where(
                    lane0, jnp.full((_L,), val, jnp.float32), wnd)
            return 0
        lax.fori_loop(kstart, _MAX_DET, _patch, 0)
        pltpu.sync_copy(out_v, out_hbm)


def kernel(boxes, scores):
    n = boxes.shape[0]
    nch = -(-n // _CH)
    npad = nch * _CH
    nbuf = npad + _L
    pad = nbuf - n

    s = jnp.where(scores > _SCORE_THRESH, scores,
                  jnp.float32(_NEG)).astype(jnp.float32)
    neg_sorted, order = lax.sort((-s, lax.iota(jnp.int32, n)),
                                 num_keys=1, is_stable=True)
    s_p = jnp.pad(-neg_sorted, (0, pad), constant_values=_NEG)
    o_p = jnp.pad(order, (0, pad))
    x1_p = jnp.pad(boxes[:, 0], (0, pad))
    y1_p = jnp.pad(boxes[:, 1], (0, pad))
    x2_p = jnp.pad(boxes[:, 2], (0, pad))
    y2_p = jnp.pad(boxes[:, 3], (0, pad))

    mesh = plsc.VectorSubcoreMesh(core_axis_name="c", subcore_axis_name="s")
    run = pl.kernel(
        functools.partial(_nms_body, nbuf, nch),
        out_type=jax.ShapeDtypeStruct((5, _CAP), jnp.float32),
        mesh=mesh,
        scratch_types=[
            pltpu.VMEM((nbuf,), jnp.float32),
            pltpu.VMEM((nbuf,), jnp.int32),
            pltpu.VMEM((nbuf,), jnp.float32),
            pltpu.VMEM((nbuf,), jnp.float32),
            pltpu.VMEM((nbuf,), jnp.float32),
            pltpu.VMEM((nbuf,), jnp.float32),
            pltpu.VMEM((_CAP,), jnp.float32),
            pltpu.VMEM((_CAP,), jnp.float32),
            pltpu.VMEM((_CAP,), jnp.float32),
            pltpu.VMEM((_CAP,), jnp.float32),
            pltpu.VMEM((_CAP,), jnp.float32),
            pltpu.VMEM((_CAP,), jnp.float32),
            pltpu.VMEM((_CAP,), jnp.float32),
            pltpu.VMEM((_CAP,), jnp.float32),
            pltpu.VMEM((_CAP,), jnp.float32),
            pltpu.VMEM((_CAP,), jnp.float32),
            pltpu.VMEM((5, _CAP), jnp.float32),
            pltpu.SMEM((4,), jnp.int32),
        ],
    )
    out = run(s_p, o_p, x1_p, y1_p, x2_p, y2_p)
    return out[:, :_MAX_DET].T


# speedup vs baseline: 1.1018x; 1.1018x over previous
import functools

import jax
import jax.numpy as jnp
from jax import lax
from jax.experimental import pallas as pl
from jax.experimental.pallas import tpu as pltpu
from jax.experimental.pallas import tpu_sc as plsc

_MAX_DET = 100
_IOU_THRESH = 0.5
_SCORE_THRESH = 0.05
_L = 16
_CAP = 128
_GATE = 112
_CH = 128
_NEG = -1e9


def _hmax16(w):
    m01 = jnp.maximum(w[0], w[1])
    m23 = jnp.maximum(w[2], w[3])
    m45 = jnp.maximum(w[4], w[5])
    m67 = jnp.maximum(w[6], w[7])
    m89 = jnp.maximum(w[8], w[9])
    mab = jnp.maximum(w[10], w[11])
    mcd = jnp.maximum(w[12], w[13])
    mef = jnp.maximum(w[14], w[15])
    return jnp.maximum(
        jnp.maximum(jnp.maximum(m01, m23), jnp.maximum(m45, m67)),
        jnp.maximum(jnp.maximum(m89, mab), jnp.maximum(mcd, mef)))


def _nms_body(nbuf, nch, p_hbm, o_hbm, out_hbm,
              p_v, o_v,
              sx1, sy1, sx2, sy2, sar, ssc,
              fx1, fy1, fx2, fy2, out_v, st):
    ox1, oy1, ox2, oy2 = nbuf, 2 * nbuf, 3 * nbuf, 4 * nbuf

    @pl.when((lax.axis_index("c") == 0) & (lax.axis_index("s") == 0))
    def _tile0():
        pltpu.sync_copy(p_hbm, p_v)
        pltpu.sync_copy(o_hbm, o_v)

        lane = lax.iota(jnp.int32, _L)
        lane0 = lane == 0
        zeros = jnp.zeros((_L,), jnp.float32)

        def _sinit(c, _):
            sl = pl.ds(c * _L, _L)
            sx1[sl] = jnp.full((_L,), -4.0, jnp.float32)
            sy1[sl] = jnp.full((_L,), -4.0, jnp.float32)
            sx2[sl] = jnp.full((_L,), -2.0, jnp.float32)
            sy2[sl] = jnp.full((_L,), -2.0, jnp.float32)
            sar[sl] = jnp.full((_L,), 4.0, jnp.float32)
            ssc[sl] = jnp.full((_L,), _NEG, jnp.float32)
            return 0
        lax.fori_loop(0, _CAP // _L, _sinit, 0)

        st[0] = jnp.int32(0)
        st[1] = jnp.int32(0)
        st[2] = jnp.int32(0)

        def _chunk(b, _):
            @pl.when(st[2] == 0)
            def _run_chunk():
                def _cand(j, c):
                    kept, fc = c
                    i = b * _CH + j
                    sc = p_v[pl.ds(i, _L)][0]
                    idx = o_v[pl.ds(i, _L)][0]
                    bx1 = p_v[pl.ds(ox1 + idx, _L)][0]
                    by1 = p_v[pl.ds(oy1 + idx, _L)][0]
                    bx2 = p_v[pl.ds(ox2 + idx, _L)][0]
                    by2 = p_v[pl.ds(oy2 + idx, _L)][0]
                    scv = jnp.full((_L,), sc, jnp.float32)
                    bx1v = jnp.full((_L,), bx1, jnp.float32)
                    by1v = jnp.full((_L,), by1, jnp.float32)
                    bx2v = jnp.full((_L,), bx2, jnp.float32)
                    by2v = jnp.full((_L,), by2, jnp.float32)
                    bav = (bx2v - bx1v) * (by2v - by1v)

                    def _iou_chunk(cc, acc):
                        sl = pl.ds(cc * _L, _L)
                        ltx = jnp.maximum(sx1[sl], bx1v)
                        lty = jnp.maximum(sy1[sl], by1v)
                        rbx = jnp.minimum(sx2[sl], bx2v)
                        rby = jnp.minimum(sy2[sl], by2v)
                        w = jnp.maximum(rbx - ltx, 0.0)
                        h = jnp.maximum(rby - lty, 0.0)
                        inter = w * h
                        iou = inter / (sar[sl] + bav - inter + 1e-9)
                        return jnp.maximum(acc, iou)
                    acc = lax.fori_loop(0, (kept + _L - 1) // _L,
                                        _iou_chunk, zeros)

                    valid = sc > jnp.float32(-1e8)
                    suppressed = _hmax16(acc) > jnp.float32(_IOU_THRESH)
                    keep_it = valid & jnp.logical_not(suppressed) & (kept < _GATE)
                    fill_it = ((jnp.logical_not(valid) | suppressed)
                               & (fc < _MAX_DET))

                    kf = jnp.where(keep_it, jnp.float32(1.0), jnp.float32(0.0))
                    kmask = jnp.where(lane0, jnp.full((_L,), kf, jnp.float32),
                                      zeros) > 0.0
                    for arr, val in ((sx1, bx1v), (sy1, by1v), (sx2, bx2v),
                                     (sy2, by2v), (sar, bav), (ssc, scv)):
                        wnd = arr[pl.ds(kept, _L)]
                        arr[pl.ds(kept, _L)] = jnp.where(kmask, val, wnd)

                    for arr, val in ((fx1, bx1v), (fy1, by1v), (fx2, bx2v),
                                     (fy2, by2v)):
                        arr[pl.ds(fc, _L)] = val

                    kept2 = kept + jnp.where(keep_it, 1, 0).astype(jnp.int32)
                    fc2 = fc + jnp.where(fill_it, 1, 0).astype(jnp.int32)
                    return (kept2, fc2)

                kept, fc = lax.fori_loop(0, _CH, _cand, (st[0], st[1]))
                st[0] = kept
                st[1] = fc
                lastsc = p_v[pl.ds(b * _CH + _CH - 1, _L)][0]
                done = ((kept >= _MAX_DET)
                        | ((lastsc <= jnp.float32(-1e8)) & (fc >= _MAX_DET)))
                st[2] = jnp.where(done, 1, 0).astype(jnp.int32)
            return 0
        lax.fori_loop(0, nch, _chunk, 0)

        def _bulk(c, _):
            sl = pl.ds(c * _L, _L)
            out_v[0, sl] = sx1[sl]
            out_v[1, sl] = sy1[sl]
            out_v[2, sl] = sx2[sl]
            out_v[3, sl] = sy2[sl]
            out_v[4, sl] = ssc[sl]
            return 0
        lax.fori_loop(0, _CAP // _L, _bulk, 0)

        kstart = jnp.minimum(st[0], jnp.int32(_MAX_DET))

        def _patch(j, _):
            kept = st[0]
            fj = j - kept
            vx1 = fx1[pl.ds(fj, _L)][0]
            vy1 = fy1[pl.ds(fj, _L)][0]
            vx2 = fx2[pl.ds(fj, _L)][0]
            vy2 = fy2[pl.ds(fj, _L)][0]
            for r, val in enumerate((vx1, vy1, vx2, vy2,
                                     jnp.float32(_NEG))):
                wnd = out_v[r, pl.ds(j, _L)]
                out_v[r, pl.ds(j, _L)] = jnp.where(
                    lane0, jnp.full((_L,), val, jnp.float32), wnd)
            return 0
        lax.fori_loop(kstart, _MAX_DET, _patch, 0)
        pltpu.sync_copy(out_v, out_hbm)


def kernel(boxes, scores):
    n = boxes.shape[0]
    nch = -(-n // _CH)
    npad = nch * _CH
    nbuf = npad + _L
    pad = nbuf - n

    s = jnp.where(scores > _SCORE_THRESH, scores,
                  jnp.float32(_NEG)).astype(jnp.float32)
    neg_sorted, order = lax.sort((-s, lax.iota(jnp.int32, n)),
                                 num_keys=1, is_stable=True)
    s_p = jnp.pad(-neg_sorted, (0, pad), constant_values=_NEG)
    o_p = jnp.pad(order, (0, pad))
    packed = jnp.concatenate([
        s_p,
        jnp.pad(boxes[:, 0], (0, pad)),
        jnp.pad(boxes[:, 1], (0, pad)),
        jnp.pad(boxes[:, 2], (0, pad)),
        jnp.pad(boxes[:, 3], (0, pad)),
    ])

    mesh = plsc.VectorSubcoreMesh(core_axis_name="c", subcore_axis_name="s")
    run = pl.kernel(
        functools.partial(_nms_body, nbuf, nch),
        out_type=jax.ShapeDtypeStruct((5, _CAP), jnp.float32),
        mesh=mesh,
        scratch_types=[
            pltpu.VMEM((5 * nbuf,), jnp.float32),
            pltpu.VMEM((nbuf,), jnp.int32),
            pltpu.VMEM((_CAP,), jnp.float32),
            pltpu.VMEM((_CAP,), jnp.float32),
            pltpu.VMEM((_CAP,), jnp.float32),
            pltpu.VMEM((_CAP,), jnp.float32),
            pltpu.VMEM((_CAP,), jnp.float32),
            pltpu.VMEM((_CAP,), jnp.float32),
            pltpu.VMEM((_CAP,), jnp.float32),
            pltpu.VMEM((_CAP,), jnp.float32),
            pltpu.VMEM((_CAP,), jnp.float32),
            pltpu.VMEM((_CAP,), jnp.float32),
            pltpu.VMEM((5, _CAP), jnp.float32),
            pltpu.SMEM((4,), jnp.int32),
        ],
    )
    out = run(packed, o_p)
    return out[:, :_MAX_DET].T


# speedup vs baseline: 1.1612x; 1.0540x over previous
import functools

import jax
import jax.numpy as jnp
from jax import lax
from jax.experimental import pallas as pl
from jax.experimental.pallas import tpu as pltpu
from jax.experimental.pallas import tpu_sc as plsc

_MAX_DET = 100
_IOU_THRESH = 0.5
_SCORE_THRESH = 0.05
_L = 16
_CAP = 128
_GATE = 112
_CH = 128
_NEG = -1e9


def _hmax16(w):
    m01 = jnp.maximum(w[0], w[1])
    m23 = jnp.maximum(w[2], w[3])
    m45 = jnp.maximum(w[4], w[5])
    m67 = jnp.maximum(w[6], w[7])
    m89 = jnp.maximum(w[8], w[9])
    mab = jnp.maximum(w[10], w[11])
    mcd = jnp.maximum(w[12], w[13])
    mef = jnp.maximum(w[14], w[15])
    return jnp.maximum(
        jnp.maximum(jnp.maximum(m01, m23), jnp.maximum(m45, m67)),
        jnp.maximum(jnp.maximum(m89, mab), jnp.maximum(mcd, mef)))


def _nms_body(nbuf, nch, p_hbm, out_hbm,
              p_v,
              sx1, sy1, sx2, sy2, sar, ssc,
              fx1, fy1, fx2, fy2, out_v, st):
    oord, ox1, oy1, ox2, oy2 = nbuf, 2 * nbuf, 3 * nbuf, 4 * nbuf, 5 * nbuf

    @pl.when((lax.axis_index("c") == 0) & (lax.axis_index("s") == 0))
    def _tile0():
        pltpu.sync_copy(p_hbm, p_v)

        lane = lax.iota(jnp.int32, _L)
        lane0 = lane == 0
        zeros = jnp.zeros((_L,), jnp.float32)

        def _sinit(c, _):
            sl = pl.ds(c * _L, _L)
            sx1[sl] = jnp.full((_L,), -4.0, jnp.float32)
            sy1[sl] = jnp.full((_L,), -4.0, jnp.float32)
            sx2[sl] = jnp.full((_L,), -2.0, jnp.float32)
            sy2[sl] = jnp.full((_L,), -2.0, jnp.float32)
            sar[sl] = jnp.full((_L,), 4.0, jnp.float32)
            ssc[sl] = jnp.full((_L,), _NEG, jnp.float32)
            return 0
        lax.fori_loop(0, _CAP // _L, _sinit, 0)

        st[0] = jnp.int32(0)
        st[1] = jnp.int32(0)
        st[2] = jnp.int32(0)

        def _chunk(b, _):
            @pl.when(st[2] == 0)
            def _run_chunk():
                def _cand(j, c):
                    kept, fc = c
                    i = b * _CH + j
                    sc = p_v[pl.ds(i, _L)][0]
                    idx = p_v[pl.ds(oord + i, _L)][0].astype(jnp.int32)
                    bx1 = p_v[pl.ds(ox1 + idx, _L)][0]
                    by1 = p_v[pl.ds(oy1 + idx, _L)][0]
                    bx2 = p_v[pl.ds(ox2 + idx, _L)][0]
                    by2 = p_v[pl.ds(oy2 + idx, _L)][0]
                    scv = jnp.full((_L,), sc, jnp.float32)
                    bx1v = jnp.full((_L,), bx1, jnp.float32)
                    by1v = jnp.full((_L,), by1, jnp.float32)
                    bx2v = jnp.full((_L,), bx2, jnp.float32)
                    by2v = jnp.full((_L,), by2, jnp.float32)
                    bav = (bx2v - bx1v) * (by2v - by1v)

                    def _iou_chunk(cc, acc):
                        sl = pl.ds(cc * _L, _L)
                        ltx = jnp.maximum(sx1[sl], bx1v)
                        lty = jnp.maximum(sy1[sl], by1v)
                        rbx = jnp.minimum(sx2[sl], bx2v)
                        rby = jnp.minimum(sy2[sl], by2v)
                        w = jnp.maximum(rbx - ltx, 0.0)
                        h = jnp.maximum(rby - lty, 0.0)
                        inter = w * h
                        iou = inter / (sar[sl] + bav - inter + 1e-9)
                        return jnp.maximum(acc, iou)
                    acc = lax.fori_loop(0, (kept + _L - 1) // _L,
                                        _iou_chunk, zeros)

                    valid = sc > jnp.float32(-1e8)
                    suppressed = _hmax16(acc) > jnp.float32(_IOU_THRESH)
                    keep_it = valid & jnp.logical_not(suppressed) & (kept < _GATE)
                    fill_it = ((jnp.logical_not(valid) | suppressed)
                               & (fc < _MAX_DET))

                    kf = jnp.where(keep_it, jnp.float32(1.0), jnp.float32(0.0))
                    kmask = jnp.where(lane0, jnp.full((_L,), kf, jnp.float32),
                                      zeros) > 0.0
                    for arr, val in ((sx1, bx1v), (sy1, by1v), (sx2, bx2v),
                                     (sy2, by2v), (sar, bav), (ssc, scv)):
                        wnd = arr[pl.ds(kept, _L)]
                        arr[pl.ds(kept, _L)] = jnp.where(kmask, val, wnd)

                    for arr, val in ((fx1, bx1v), (fy1, by1v), (fx2, bx2v),
                                     (fy2, by2v)):
                        arr[pl.ds(fc, _L)] = val

                    kept2 = kept + jnp.where(keep_it, 1, 0).astype(jnp.int32)
                    fc2 = fc + jnp.where(fill_it, 1, 0).astype(jnp.int32)
                    return (kept2, fc2)

                kept, fc = lax.fori_loop(0, _CH, _cand, (st[0], st[1]))
                st[0] = kept
                st[1] = fc
                lastsc = p_v[pl.ds(b * _CH + _CH - 1, _L)][0]
                done = ((kept >= _MAX_DET)
                        | ((lastsc <= jnp.float32(-1e8)) & (fc >= _MAX_DET)))
                st[2] = jnp.where(done, 1, 0).astype(jnp.int32)
            return 0
        lax.fori_loop(0, nch, _chunk, 0)

        def _bulk(c, _):
            sl = pl.ds(c * _L, _L)
            out_v[0, sl] = sx1[sl]
            out_v[1, sl] = sy1[sl]
            out_v[2, sl] = sx2[sl]
            out_v[3, sl] = sy2[sl]
            out_v[4, sl] = ssc[sl]
            return 0
        lax.fori_loop(0, _CAP // _L, _bulk, 0)

        kstart = jnp.minimum(st[0], jnp.int32(_MAX_DET))

        def _patch(j, _):
            kept = st[0]
            fj = j - kept
            vx1 = fx1[pl.ds(fj, _L)][0]
            vy1 = fy1[pl.ds(fj, _L)][0]
            vx2 = fx2[pl.ds(fj, _L)][0]
            vy2 = fy2[pl.ds(fj, _L)][0]
            for r, val in enumerate((vx1, vy1, vx2, vy2,
                                     jnp.float32(_NEG))):
                wnd = out_v[r, pl.ds(j, _L)]
                out_v[r, pl.ds(j, _L)] = jnp.where(
                    lane0, jnp.full((_L,), val, jnp.float32), wnd)
            return 0
        lax.fori_loop(kstart, _MAX_DET, _patch, 0)
        pltpu.sync_copy(out_v, out_hbm)


def kernel(boxes, scores):
    n = boxes.shape[0]
    nch = -(-n // _CH)
    npad = nch * _CH
    nbuf = npad + _L
    pad = nbuf - n

    s = jnp.where(scores > _SCORE_THRESH, scores,
                  jnp.float32(_NEG)).astype(jnp.float32)
    neg_sorted, order = lax.sort((-s, lax.iota(jnp.int32, n)),
                                 num_keys=1, is_stable=True)
    s_p = jnp.pad(-neg_sorted, (0, pad), constant_values=_NEG)
    packed = jnp.concatenate([
        s_p,
        jnp.pad(order.astype(jnp.float32), (0, pad)),
        jnp.pad(boxes[:, 0], (0, pad)),
        jnp.pad(boxes[:, 1], (0, pad)),
        jnp.pad(boxes[:, 2], (0, pad)),
        jnp.pad(boxes[:, 3], (0, pad)),
    ])

    mesh = plsc.VectorSubcoreMesh(core_axis_name="c", subcore_axis_name="s")
    run = pl.kernel(
        functools.partial(_nms_body, nbuf, nch),
        out_type=jax.ShapeDtypeStruct((5, _CAP), jnp.float32),
        mesh=mesh,
        scratch_types=[
            pltpu.VMEM((6 * nbuf,), jnp.float32),
            pltpu.VMEM((_CAP,), jnp.float32),
            pltpu.VMEM((_CAP,), jnp.float32),
            pltpu.VMEM((_CAP,), jnp.float32),
            pltpu.VMEM((_CAP,), jnp.float32),
            pltpu.VMEM((_CAP,), jnp.float32),
            pltpu.VMEM((_CAP,), jnp.float32),
            pltpu.VMEM((_CAP,), jnp.float32),
            pltpu.VMEM((_CAP,), jnp.float32),
            pltpu.VMEM((_CAP,), jnp.float32),
            pltpu.VMEM((_CAP,), jnp.float32),
            pltpu.VMEM((5, _CAP), jnp.float32),
            pltpu.SMEM((4,), jnp.int32),
        ],
    )
    out = run(packed)
    return out[:, :_MAX_DET].T
